# baseline (speedup 1.0000x reference)
import functools

import jax
import jax.numpy as jnp
from jax import lax
from jax.experimental import pallas as pl
from jax.experimental.pallas import tpu as pltpu
from jax.experimental.pallas import tpu_sc as plsc

_B = 16384
_D = 32
_NE = 1000000
_NR = 1000
_NC = 2
_NS = 16
_NW = _NC * _NS
_BW = _B // _NW


def _rsqrt(x):
    i = plsc.bitcast(x, jnp.int32)
    i = jnp.int32(0x5F3759DF) - lax.shift_right_logical(i, 1)
    y = plsc.bitcast(i, jnp.float32)
    for _ in range(3):
        y = y * (1.5 - 0.5 * x * y * y)
    return y


def _body(head_hbm, rel_hbm, tail_hbm, etab_hbm, rtab_hbm, out_hbm,
          hidx, tidx, ridx,
          hcols, tcols, rcols, scores, sem):
    wid = lax.axis_index("s") * _NC + lax.axis_index("c")
    base = wid * _BW

    pltpu.sync_copy(head_hbm.at[pl.ds(base, _BW)], hidx)
    pltpu.sync_copy(tail_hbm.at[pl.ds(base, _BW)], tidx)
    pltpu.sync_copy(rel_hbm.at[pl.ds(base, _BW)], ridx)

    cps = []
    for j in range(_D):
        sl = pl.ds(j * _BW, _BW)
        cps.append(pltpu.async_copy(etab_hbm.at[j].at[hidx], hcols.at[sl], sem))
        cps.append(pltpu.async_copy(etab_hbm.at[j].at[tidx], tcols.at[sl], sem))
        cps.append(pltpu.async_copy(rtab_hbm.at[j].at[ridx], rcols.at[sl], sem))
    for cp in cps:
        cp.wait()

    def group(g, carry):
        sl = pl.ds(g * 16, 16)
        z = jnp.zeros((16,), jnp.float32)
        hh = tt = rr = hr = tr = ht = z
        for j in range(_D):
            csl = pl.ds(j * _BW + g * 16, 16)
            hj = hcols[csl]
            tj = tcols[csl]
            rj = rcols[csl]
            hh = hh + hj * hj
            tt = tt + tj * tj
            rr = rr + rj * rj
            hr = hr + hj * rj
            tr = tr + tj * rj
            ht = ht + hj * tj
        one = jnp.ones((16,), jnp.float32)
        sh = jnp.where(hh > 1.0, _rsqrt(hh), one)
        st = jnp.where(tt > 1.0, _rsqrt(tt), one)
        s = (sh * sh * hh + st * st * tt + rr
             + 2.0 * (sh * hr) - 2.0 * (st * tr) - 2.0 * (sh * (st * ht)))
        s = jnp.maximum(s, 0.0)
        score = jnp.where(s > 0.0, -(s * _rsqrt(s)), z)
        scores[sl] = score
        return carry

    lax.fori_loop(0, _BW // 16, group, 0)
    pltpu.sync_copy(scores, out_hbm.at[pl.ds(base, _BW)])


_transe_sc = functools.partial(
    pl.kernel,
    out_type=jax.ShapeDtypeStruct((_B,), jnp.float32),
    mesh=plsc.VectorSubcoreMesh(core_axis_name="c", subcore_axis_name="s"),
    compiler_params=pltpu.CompilerParams(
        needs_layout_passes=False, use_tc_tiling_on_sc=False),
    scratch_types=[
        pltpu.VMEM((_BW,), jnp.int32),
        pltpu.VMEM((_BW,), jnp.int32),
        pltpu.VMEM((_BW,), jnp.int32),
        pltpu.VMEM((_D * _BW,), jnp.float32),
        pltpu.VMEM((_D * _BW,), jnp.float32),
        pltpu.VMEM((_D * _BW,), jnp.float32),
        pltpu.VMEM((_BW,), jnp.float32),
        pltpu.SemaphoreType.DMA,
    ],
)(_body)


def kernel(head_ids, rel_ids, tail_ids, entity_table, relation_table):
    return _transe_sc(head_ids.astype(jnp.int32), rel_ids.astype(jnp.int32),
                      tail_ids.astype(jnp.int32), entity_table.T,
                      relation_table.T)


# speedup vs baseline: 4.9837x; 4.9837x over previous
import functools

import jax
import jax.numpy as jnp
from jax import lax
from jax.experimental import pallas as pl
from jax.experimental.pallas import tpu as pltpu
from jax.experimental.pallas import tpu_sc as plsc

_B = 16384
_D = 32
_NE = 1000000
_NR = 1000
_NC = 2
_NS = 16
_NW = _NC * _NS
_BW = _B // _NW
_CHUNK = 128
_HALF = 256


def _rsqrt(x):
    i = plsc.bitcast(x, jnp.int32)
    i = jnp.int32(0x5F3759DF) - lax.shift_right_logical(i, 1)
    y = plsc.bitcast(i, jnp.float32)
    for _ in range(3):
        y = y * (1.5 - 0.5 * x * y * y)
    return y


def _body(head_hbm, rel_hbm, tail_hbm, etab_hbm, rtab_hbm, out_hbm,
          hidx, tidx, ridx, hrows, trows, rrows, scores, sem):
    wid = lax.axis_index("s") * _NC + lax.axis_index("c")
    base = wid * _BW

    pltpu.sync_copy(head_hbm.at[pl.ds(base, _BW)], hidx)
    pltpu.sync_copy(tail_hbm.at[pl.ds(base, _BW)], tidx)
    pltpu.sync_copy(rel_hbm.at[pl.ds(base, _BW)], ridx)

    lane = lax.iota(jnp.int32, 16)

    for h in range(_BW // _HALF):
        cps = []
        for c in range(_HALF // _CHUNK):
            isl = pl.ds(h * _HALF + c * _CHUNK, _CHUNK)
            dsl = pl.ds(c * _CHUNK, _CHUNK)
            cps.append(pltpu.async_copy(etab_hbm.at[hidx.at[isl]],
                                        hrows.at[dsl], sem))
            cps.append(pltpu.async_copy(etab_hbm.at[tidx.at[isl]],
                                        trows.at[dsl], sem))
            cps.append(pltpu.async_copy(rtab_hbm.at[ridx.at[isl]],
                                        rrows.at[dsl], sem))
        for cp in cps:
            cp.wait()

        def block(b, carry):
            rvec = b * 16 + lane
            sl = pl.ds(h * _HALF + b * 16, 16)
            z = jnp.zeros((16,), jnp.float32)
            hh = tt = rr = hr = tr = ht = z
            for j in range(_D):
                col = jnp.full((16,), j, jnp.int32)
                hj = plsc.load_gather(hrows, [rvec, col])
                tj = plsc.load_gather(trows, [rvec, col])
                rj = plsc.load_gather(rrows, [rvec, col])
                hh = hh + hj * hj
                tt = tt + tj * tj
                rr = rr + rj * rj
                hr = hr + hj * rj
                tr = tr + tj * rj
                ht = ht + hj * tj
            one = jnp.ones((16,), jnp.float32)
            sh = jnp.where(hh > 1.0, _rsqrt(hh), one)
            st = jnp.where(tt > 1.0, _rsqrt(tt), one)
            s = (sh * sh * hh + st * st * tt + rr
                 + 2.0 * (sh * hr) - 2.0 * (st * tr) - 2.0 * (sh * (st * ht)))
            s = jnp.maximum(s, 0.0)
            score = jnp.where(s > 0.0, -(s * _rsqrt(s)), z)
            scores[sl] = score
            return carry

        lax.fori_loop(0, _HALF // 16, block, 0)

    pltpu.sync_copy(scores, out_hbm.at[pl.ds(base, _BW)])


_transe_sc = functools.partial(
    pl.kernel,
    out_type=jax.ShapeDtypeStruct((_B,), jnp.float32),
    mesh=plsc.VectorSubcoreMesh(core_axis_name="c", subcore_axis_name="s"),
    compiler_params=pltpu.CompilerParams(
        needs_layout_passes=False, use_tc_tiling_on_sc=True),
    scratch_types=[
        pltpu.VMEM((_BW,), jnp.int32),
        pltpu.VMEM((_BW,), jnp.int32),
        pltpu.VMEM((_BW,), jnp.int32),
        pltpu.VMEM((_HALF, 128), jnp.float32),
        pltpu.VMEM((_HALF, 128), jnp.float32),
        pltpu.VMEM((_HALF, 128), jnp.float32),
        pltpu.VMEM((_BW,), jnp.float32),
        pltpu.SemaphoreType.DMA,
    ],
)(_body)


def kernel(head_ids, rel_ids, tail_ids, entity_table, relation_table):
    etab = jnp.pad(entity_table, ((0, 0), (0, 128 - _D)))
    rtab = jnp.pad(relation_table, ((0, 0), (0, 128 - _D)))
    return _transe_sc(head_ids.astype(jnp.int32), rel_ids.astype(jnp.int32),
                      tail_ids.astype(jnp.int32), etab, rtab)
